# Initial kernel scaffold; baseline (speedup 1.0000x reference)
import functools

import jax
import jax.numpy as jnp
from jax import lax
from jax.experimental import pallas as pl
from jax.experimental.pallas import tpu as pltpu
from jax.experimental.pallas import tpu_sc as plsc

N_USER = 1000000
N_ITEM = 1000000
EMB = 64
BATCH = 4096

_info = plsc.get_sparse_core_info()
_NC, _NS = _info.num_cores, _info.num_subcores
_NW = _NC * _NS
_BPW = BATCH // _NW


@functools.partial(
    pl.kernel,
    mesh=plsc.VectorSubcoreMesh(core_axis_name="c", subcore_axis_name="s"),
    out_type=(
        jax.ShapeDtypeStruct((BATCH, EMB), jnp.float32),
        jax.ShapeDtypeStruct((BATCH, EMB), jnp.float32),
    ),
    scratch_types=[
        pltpu.VMEM((_BPW,), jnp.int32),
        pltpu.VMEM((_BPW, EMB), jnp.float32),
        pltpu.VMEM((_BPW,), jnp.int32),
        pltpu.VMEM((_BPW, EMB), jnp.float32),
        pltpu.SemaphoreType.DMA,
        pltpu.SemaphoreType.DMA,
    ],
)
def _sc_gather(users_hbm, items_hbm, utab_hbm, itab_hbm,
               uout_hbm, iout_hbm,
               uidx_v, urows_v, iidx_v, irows_v, usem, isem):
    wid = lax.axis_index("s") * _NC + lax.axis_index("c")
    base = wid * _BPW
    pltpu.sync_copy(users_hbm.at[pl.ds(base, _BPW)], uidx_v)
    pltpu.sync_copy(items_hbm.at[pl.ds(base, _BPW)], iidx_v)
    cu = pltpu.async_copy(utab_hbm.at[uidx_v], urows_v, usem)
    ci = pltpu.async_copy(itab_hbm.at[iidx_v], irows_v, isem)
    cu.wait()
    ci.wait()
    pltpu.sync_copy(urows_v, uout_hbm.at[pl.ds(base, _BPW)])
    pltpu.sync_copy(irows_v, iout_hbm.at[pl.ds(base, _BPW)])


def _mm_body(u_ref, i_ref, o_ref):
    o_ref[...] = lax.dot_general(
        u_ref[...], i_ref[...],
        dimension_numbers=(((1,), (1,)), ((), ())),
        preferred_element_type=jnp.float32,
    )


_BM = 512


def kernel(users, pos_items, user_table, item_table):
    u_emb, i_emb = _sc_gather(users, pos_items, user_table, item_table)
    score = pl.pallas_call(
        _mm_body,
        grid=(BATCH // _BM,),
        in_specs=[
            pl.BlockSpec((_BM, EMB), lambda i: (i, 0)),
            pl.BlockSpec((BATCH, EMB), lambda i: (0, 0)),
        ],
        out_specs=pl.BlockSpec((_BM, BATCH), lambda i: (i, 0)),
        out_shape=jax.ShapeDtypeStruct((BATCH, BATCH), jnp.float32),
    )(u_emb, i_emb)
    return score


# speedup vs baseline: 1.2021x; 1.2021x over previous
import jax
import jax.numpy as jnp
from jax import lax
from jax.experimental import pallas as pl

EMB = 64
BATCH = 4096
_BM = 512


def _mm_body(u_ref, i_ref, o_ref):
    o_ref[...] = lax.dot_general(
        u_ref[...], i_ref[...],
        dimension_numbers=(((1,), (1,)), ((), ())),
        preferred_element_type=jnp.float32,
    )


def kernel(users, pos_items, user_table, item_table):
    u_emb = jnp.take(user_table, users, axis=0)
    i_emb = jnp.take(item_table, pos_items, axis=0)
    score = pl.pallas_call(
        _mm_body,
        grid=(BATCH // _BM,),
        in_specs=[
            pl.BlockSpec((_BM, EMB), lambda i: (i, 0)),
            pl.BlockSpec((BATCH, EMB), lambda i: (0, 0)),
        ],
        out_specs=pl.BlockSpec((_BM, BATCH), lambda i: (i, 0)),
        out_shape=jax.ShapeDtypeStruct((BATCH, BATCH), jnp.float32),
    )(u_emb, i_emb)
    return score


# speedup vs baseline: 3.3203x; 2.7622x over previous
import functools

import jax
import jax.numpy as jnp
from jax import lax
from jax.experimental import pallas as pl
from jax.experimental.pallas import tpu as pltpu
from jax.experimental.pallas import tpu_sc as plsc

EMB = 64
BATCH = 4096
NROW = 1000000

_info = plsc.get_sparse_core_info()
_NC, _NS = _info.num_cores, _info.num_subcores
_NW = _NC * _NS
_BPW = BATCH // _NW
_K = 8
_MAXOFF = NROW - 128


def _lane(vec, l):
    return lax.squeeze(lax.slice(vec, (l,), (l + 1,)), (0,))


_SC_KERNEL_KWARGS = dict(
    mesh=plsc.VectorSubcoreMesh(core_axis_name="c", subcore_axis_name="s"),
    compiler_params=pltpu.CompilerParams(needs_layout_passes=False),
    out_type=(
        jax.ShapeDtypeStruct((BATCH * EMB,), jnp.float32),
        jax.ShapeDtypeStruct((BATCH * EMB,), jnp.float32),
    ),
    scratch_types=[
        pltpu.VMEM((BATCH,), jnp.int32),
        pltpu.VMEM((BATCH,), jnp.int32),
        pltpu.VMEM((_BPW + 16,), jnp.int32),
        pltpu.VMEM((_BPW + 16,), jnp.int32),
        pltpu.VMEM((_BPW + 16,), jnp.int32),
        pltpu.VMEM((_BPW + 16,), jnp.int32),
        pltpu.VMEM((8, 8, 128), jnp.float32),
        pltpu.VMEM((8, 8, 128), jnp.float32),
        pltpu.VMEM((8, 8, 128), jnp.float32),
        pltpu.VMEM((8, 8, 128), jnp.float32),
        pltpu.VMEM((8, 8, 128), jnp.float32),
        pltpu.VMEM((8, 8, 128), jnp.float32),
        pltpu.VMEM((8, 8, 128), jnp.float32),
        pltpu.VMEM((8, 8, 128), jnp.float32),
        pltpu.VMEM((_BPW * EMB,), jnp.float32),
        pltpu.VMEM((_BPW * EMB,), jnp.float32),
        pltpu.SemaphoreType.DMA,
        pltpu.SemaphoreType.DMA,
        pltpu.SemaphoreType.DMA,
        pltpu.SemaphoreType.DMA,
        pltpu.SemaphoreType.DMA,
        pltpu.SemaphoreType.DMA,
        pltpu.SemaphoreType.DMA,
        pltpu.SemaphoreType.DMA,
        pltpu.SemaphoreType.DMA,
    ],
)


def _sc_gather_body(users_hbm, items_hbm, utab_hbm, itab_hbm,
               uout_hbm, iout_hbm,
               uidx_v, iidx_v,
               uoff_v, uln_v, ioff_v, iln_v,
               stg0, stg1, stg2, stg3, stg4, stg5, stg6, stg7,
               uout_v, iout_v,
               sem0, sem1, sem2, sem3, sem4, sem5, sem6, sem7, osem):
    sems = (sem0, sem1, sem2, sem3, sem4, sem5, sem6, sem7)
    stgs = (stg0, stg1, stg2, stg3, stg4, stg5, stg6, stg7)
    wid = lax.axis_index("s") * _NC + lax.axis_index("c")
    base = wid * _BPW

    pltpu.sync_copy(users_hbm, uidx_v)
    pltpu.sync_copy(items_hbm, iidx_v)
    m128 = jnp.full((16,), -128, jnp.int32)
    for k in range(_BPW // 16):
        sl = pl.ds(k * 16, 16)
        uv = uidx_v[pl.ds(base + k * 16, 16)]
        iv = iidx_v[pl.ds(base + k * 16, 16)]
        uo = lax.bitwise_and(uv, m128)
        io = lax.bitwise_and(iv, m128)
        uln_v[sl] = uv - uo
        iln_v[sl] = iv - io
        uoff_v[sl] = uo
        ioff_v[sl] = io

    avecs = [jnp.arange(c0, c0 + 16, dtype=jnp.int32) // 8
             for c0 in range(0, EMB, 16)]
    svecs = [jnp.arange(c0, c0 + 16, dtype=jnp.int32) % 8
             for c0 in range(0, EMB, 16)]

    def table_pass(tab, off_v, ln_v, out_v):
        def fire_chunk(ci):
            offv = off_v[pl.ds(ci * 8, 16)]
            for lane in range(8):
                off = pl.multiple_of(_lane(offv, lane), 128)
                for a in range(8):
                    pltpu.async_copy(
                        tab.at[a, pl.ds(0, 8), pl.ds(off, 128)],
                        stgs[lane].at[a], sems[lane])

        def extract_chunk(ci):
            lnv = ln_v[pl.ds(ci * 8, 16)]
            for lane in range(8):
                pltpu.make_async_copy(
                    tab.at[pl.ds(0, 8), pl.ds(0, 8), pl.ds(0, 128)],
                    stgs[lane], sems[lane]).wait()
                l = _lane(lnv, lane)
                lv = jnp.zeros((16,), jnp.int32) + l
                for c16 in range(EMB // 16):
                    vals = plsc.load_gather(stgs[lane],
                                            [avecs[c16], svecs[c16], lv])
                    out_v[pl.ds((ci * 8 + lane) * EMB + c16 * 16, 16)] = vals

        fire_chunk(0)

        def body(ci, carry):
            extract_chunk(ci - 1)
            fire_chunk(ci)
            return carry

        lax.fori_loop(1, _BPW // 8, body, 0)
        extract_chunk(_BPW // 8 - 1)

    table_pass(utab_hbm, uoff_v, uln_v, uout_v)
    table_pass(itab_hbm, ioff_v, iln_v, iout_v)

    cu = pltpu.async_copy(uout_v, uout_hbm.at[pl.ds(base * EMB, _BPW * EMB)],
                          osem)
    ci = pltpu.async_copy(iout_v, iout_hbm.at[pl.ds(base * EMB, _BPW * EMB)],
                          osem)
    cu.wait()
    ci.wait()


_sc_gather = pl.kernel(**_SC_KERNEL_KWARGS)(_sc_gather_body)


def _mm_body(u_ref, i_ref, o_ref):
    o_ref[...] = lax.dot_general(
        u_ref[...], i_ref[...],
        dimension_numbers=(((1,), (1,)), ((), ())),
        preferred_element_type=jnp.float32,
    )


_BM = 512


def kernel(users, pos_items, user_table, item_table):
    utab3 = user_table.T.reshape(8, 8, NROW)
    itab3 = item_table.T.reshape(8, 8, NROW)
    u_flat, i_flat = _sc_gather(users, pos_items, utab3, itab3)
    u_emb = u_flat.reshape(BATCH, EMB)
    i_emb = i_flat.reshape(BATCH, EMB)
    score = pl.pallas_call(
        _mm_body,
        grid=(BATCH // _BM,),
        in_specs=[
            pl.BlockSpec((_BM, EMB), lambda i: (i, 0)),
            pl.BlockSpec((BATCH, EMB), lambda i: (0, 0)),
        ],
        out_specs=pl.BlockSpec((_BM, BATCH), lambda i: (i, 0)),
        out_shape=jax.ShapeDtypeStruct((BATCH, BATCH), jnp.float32),
    )(u_emb, i_emb)
    return score
